# baseline (speedup 1.0000x reference)
import functools
import math

import jax
import jax.numpy as jnp
from jax import lax
from jax.experimental import pallas as pl
from jax.experimental.pallas import tpu as pltpu
from jax.experimental.pallas import tpu_sc as plsc

_DIM = 32
_NW = 32
_C = 512
_NIDX = 128
_BLK = 2048


def _make_sc_gather(n_hist, batch, scale):
    mesh = plsc.VectorSubcoreMesh(core_axis_name="c", subcore_axis_name="s")
    n_iblk = batch // _C
    assert n_iblk == _NW
    qs = _C // _NIDX

    @functools.partial(
        pl.kernel,
        mesh=mesh,
        out_type=jax.ShapeDtypeStruct(
            (n_hist, _DIM // 8, batch // 128, 8, 128), jnp.float32),
        scratch_types=[
            pltpu.VMEM((n_hist, qs, _NIDX), jnp.int32),
            pltpu.VMEM((2, _C, _DIM), jnp.float32),
            pltpu.VMEM((2, _DIM // 8, _C // 128, 8, 128), jnp.float32),
            pltpu.VMEM_SHARED((16, 2, _DIM // 8, _C // 128, 8, 128),
                              jnp.float32),
            pltpu.SemaphoreType.DMA((2,)),
            pltpu.SemaphoreType.DMA((2,)),
            pltpu.SemaphoreType.DMA((2,)),
            pltpu.SemaphoreType.DMA,
        ],
        compiler_params=pltpu.CompilerParams(
            use_tc_tiling_on_sc=False, needs_layout_passes=False),
    )
    def k(x_hbm, w_hbm, out_hbm, idx_v, gbuf, tbuf, shared, gsem, xsem,
          ssem, isem):
        sid = lax.axis_index("s")
        wid = sid * 2 + lax.axis_index("c")
        pltpu.async_copy(
            x_hbm.at[:, pl.ds(wid * qs, qs), :], idx_v, isem).wait()

        def gath(j, b, q):
            return pltpu.make_async_copy(
                w_hbm.at[idx_v.at[j, q]],
                gbuf.at[b, pl.ds(q * _NIDX, _NIDX)],
                gsem.at[b])

        def hop(b):
            return pltpu.make_async_copy(
                tbuf.at[b], shared.at[sid, b], xsem.at[b])

        def stor(j, b):
            return pltpu.make_async_copy(
                shared.at[sid, b],
                out_hbm.at[j, :, pl.ds(wid * (_C // 128), _C // 128), :, :],
                ssem.at[b])

        rows16 = lax.iota(jnp.int32, 16)

        def transpose_task(b):
            def tbody(c2, c):
                rows = c2 * 16 + rows16
                tb = c2 >> 3
                sl = pl.ds((c2 & 7) * 16, 16)
                for d in range(_DIM):
                    cols = jnp.full((16,), d, jnp.int32)
                    tbuf[b, d // 8, tb, d % 8, sl] = plsc.load_gather(
                        gbuf.at[b], [rows, cols]) * scale
                return c

            lax.fori_loop(0, _C // 16, tbody, 0)

        for b in range(2):
            for q in range(qs):
                gath(b, b, q).start()
        for b in range(2):
            for q in range(qs):
                gath(b, b, q).wait()
            transpose_task(b)
            hop(b).start()
            hop(b).wait()
            stor(b, b).start()
            for q in range(qs):
                gath(b + 2, b, q).start()

        def main_body(g, c):
            for b in range(2):
                j = g * 2 + b
                for q in range(qs):
                    gath(j, b, q).wait()
                stor(j - 2, b).wait()
                transpose_task(b)
                hop(b).start()
                hop(b).wait()
                stor(j, b).start()
                for q in range(qs):
                    gath(j + 2, b, q).start()
            return c

        lax.fori_loop(1, n_hist // 2 - 1, main_body, 0)

        for b in range(2):
            j = n_hist - 2 + b
            for q in range(qs):
                gath(j, b, q).wait()
            stor(j - 2, b).wait()
            transpose_task(b)
            hop(b).start()
            hop(b).wait()
            stor(j, b).start()
        for b in range(2):
            stor(n_hist - 2 + b, b).wait()

    return k


def kernel(x, w):
    batch, n_hist = x.shape
    scale = math.sqrt(w.shape[1])
    x3 = x.T.reshape(n_hist, batch // _NIDX, _NIDX).astype(jnp.int32)
    out5 = _make_sc_gather(n_hist, batch, scale)(x3, w)
    return jnp.transpose(out5, (2, 4, 0, 1, 3)).reshape(batch, n_hist, _DIM)


# speedup vs baseline: 1.3212x; 1.3212x over previous
import functools
import math

import jax
import jax.numpy as jnp
from jax import lax
from jax.experimental import pallas as pl
from jax.experimental.pallas import tpu as pltpu
from jax.experimental.pallas import tpu_sc as plsc

_DIM = 32
_NW = 32
_C = 512
_NIDX = 128
_BLK = 2048


def _make_sc_gather(n_hist, batch, scale):
    mesh = plsc.VectorSubcoreMesh(core_axis_name="c", subcore_axis_name="s")
    n_iblk = batch // _C
    assert n_iblk == _NW
    qs = _C // _NIDX

    @functools.partial(
        pl.kernel,
        mesh=mesh,
        out_type=jax.ShapeDtypeStruct(
            (n_hist, _DIM // 8, batch // 128, 8, 128), jnp.float32),
        scratch_types=[
            pltpu.VMEM((n_hist, qs, _NIDX), jnp.int32),
            pltpu.VMEM((2, _C, _DIM), jnp.float32),
            pltpu.VMEM((2, _DIM // 8, _C // 128, 8, 128), jnp.float32),
            pltpu.SemaphoreType.DMA((2,)),
            pltpu.SemaphoreType.DMA((2,)),
            pltpu.SemaphoreType.DMA,
        ],
        compiler_params=pltpu.CompilerParams(
            use_tc_tiling_on_sc=False, needs_layout_passes=False),
    )
    def k(x_hbm, w_hbm, out_hbm, idx_v, gbuf, tbuf, gsem, ssem, isem):
        wid = lax.axis_index("s") * 2 + lax.axis_index("c")
        i0 = wid * _C
        pltpu.async_copy(
            x_hbm.at[:, pl.ds(wid * qs, qs), :], idx_v, isem).wait()

        def gath(j, b, q):
            return pltpu.make_async_copy(
                w_hbm.at[idx_v.at[j, q]],
                gbuf.at[b, pl.ds(q * _NIDX, _NIDX)],
                gsem.at[b])

        def stor(j, b):
            return pltpu.make_async_copy(
                tbuf.at[b],
                out_hbm.at[j, :, pl.ds(wid * (_C // 128), _C // 128), :, :],
                ssem.at[b])

        rows16 = lax.iota(jnp.int32, 16)

        def transpose_task(b):
            @plsc.parallel_loop(0, _C // 16, 1, unroll=2)
            def tbody(c2):
                rows = c2 * 16 + rows16
                tb = c2 >> 3
                sl = pl.ds((c2 & 7) * 16, 16)
                for d in range(_DIM):
                    cols = jnp.full((16,), d, jnp.int32)
                    tbuf[b, d // 8, tb, d % 8, sl] = plsc.load_gather(
                        gbuf.at[b], [rows, cols]) * scale

        for b in range(2):
            for q in range(qs):
                gath(b, b, q).start()
        for b in range(2):
            for q in range(qs):
                gath(b, b, q).wait()
            transpose_task(b)
            stor(b, b).start()
            for q in range(qs):
                gath(b + 2, b, q).start()

        def main_body(g, c):
            for b in range(2):
                j = g * 2 + b
                for q in range(qs):
                    gath(j, b, q).wait()
                stor(j - 2, b).wait()
                transpose_task(b)
                stor(j, b).start()
                for q in range(qs):
                    gath(j + 2, b, q).start()
            return c

        lax.fori_loop(1, n_hist // 2 - 1, main_body, 0)

        for b in range(2):
            j = n_hist - 2 + b
            for q in range(qs):
                gath(j, b, q).wait()
            stor(j - 2, b).wait()
            transpose_task(b)
            stor(j, b).start()
        for b in range(2):
            stor(n_hist - 2 + b, b).wait()

    return k


def kernel(x, w):
    batch, n_hist = x.shape
    scale = math.sqrt(w.shape[1])
    x3 = x.T.reshape(n_hist, batch // _NIDX, _NIDX).astype(jnp.int32)
    out5 = _make_sc_gather(n_hist, batch, scale)(x3, w)
    return jnp.transpose(out5, (2, 4, 0, 1, 3)).reshape(batch, n_hist, _DIM)
